# baseline (speedup 1.0000x reference)
import functools

import jax
import jax.numpy as jnp
from jax import lax
from jax.experimental import pallas as pl
from jax.experimental.pallas import tpu as pltpu
from jax.experimental.pallas import tpu_sc as plsc

B = 16384
BH = B // 2
L = 200
VOCAB = 1000
D = 64
KP = 1024

NC = 2
NS = 16
NW = NC * NS
ROWS_PER_W = BH // NW
CHUNK = 32
NCHUNKS = ROWS_PER_W // CHUNK
NPAIR = NCHUNKS // 2

_mesh = plsc.VectorSubcoreMesh(core_axis_name="c", subcore_axis_name="s")


@functools.partial(
    pl.kernel,
    mesh=_mesh,
    out_type=jax.ShapeDtypeStruct((BH, KP), jnp.float32),
    scratch_types=[
        pltpu.VMEM((2, CHUNK, L), jnp.int32),
        pltpu.VMEM((2, CHUNK, KP), jnp.float32),
        pltpu.SemaphoreType.DMA,
        pltpu.SemaphoreType.DMA,
    ],
    compiler_params=pltpu.CompilerParams(needs_layout_passes=False),
)
def _hist(idx_hbm, cnt_hbm, idx_v, cnt_v, sem0, sem1):
    wid = lax.axis_index("s") * NC + lax.axis_index("c")
    row_base_w = wid * ROWS_PER_W
    zero16 = jnp.zeros((16,), jnp.float32)
    ones = jnp.full((16,), 1.0, jnp.float32)
    elig = lax.iota(jnp.int32, 16) >= 8
    sems = (sem0, sem1)

    _OFFS = tuple(s * 16 for s in range(L // 16)) + (L - 16,)

    def zero_old(b):
        def clear_row(i, c):
            for v in range(KP // 16):
                cnt_v[b, i, pl.ds(v * 16, 16)] = zero16
            return c

        lax.fori_loop(0, CHUNK, clear_row, 0)

    def fill_chunk(ci, b):
        base = row_base_w + ci * CHUNK
        pltpu.sync_copy(idx_hbm.at[pl.ds(base, CHUNK)], idx_v.at[b])
        bvec = jnp.full((16,), b, jnp.int32)

        def row_body(r, carry2):
            rvec = jnp.full((16,), 0, jnp.int32) + r
            for off in _OFFS[:-1]:
                ivec = idx_v[b, r, pl.ds(off, 16)]
                plsc.addupdate_scatter(cnt_v, [bvec, rvec, ivec], ones)
            ivec = idx_v[b, r, pl.ds(_OFFS[-1], 16)]
            plsc.addupdate_scatter(cnt_v, [bvec, rvec, ivec], ones,
                                   mask=elig)
            return carry2

        lax.fori_loop(0, CHUNK, row_body, 0)
        return base

    def start_out(ci, b):
        base = row_base_w + ci * CHUNK
        pltpu.async_copy(cnt_v.at[b], cnt_hbm.at[pl.ds(base, CHUNK)],
                         sems[b])

    def wait_out(ci, b):
        base = row_base_w + ci * CHUNK
        pltpu.make_async_copy(cnt_v.at[b],
                              cnt_hbm.at[pl.ds(base, CHUNK)],
                              sems[b]).wait()

    def clear_body(i, c):
        for bb in range(2):
            for v in range(16):
                cnt_v[bb, i // (KP // 256),
                      pl.ds((i % (KP // 256) * 16 + v) * 16, 16)] = zero16
        return c

    lax.fori_loop(0, CHUNK * KP // 256, clear_body, 0)

    for b in range(2):
        fill_chunk(b, b)
        start_out(b, b)

    def pair_body(g, carry):
        for b in range(2):
            ci = g * 2 + b
            wait_out(ci - 2, b)
            zero_old(b)
            fill_chunk(ci, b)
            start_out(ci, b)
        return carry

    lax.fori_loop(1, NPAIR, pair_body, 0)
    for b in range(2):
        wait_out(NCHUNKS - 2 + b, b)


TM = 2048


def _mm_body(c_ref, t_ref, o_ref):
    o_ref[...] = jnp.dot(
        c_ref[...].astype(jnp.bfloat16),
        t_ref[...],
        preferred_element_type=jnp.float32,
    )


_mm = pl.pallas_call(
    _mm_body,
    grid=(BH // TM,),
    in_specs=[
        pl.BlockSpec((TM, KP), lambda i: (i, 0)),
        pl.BlockSpec((KP, D), lambda i: (0, 0)),
    ],
    out_specs=pl.BlockSpec((TM, D), lambda i: (i, 0)),
    out_shape=jax.ShapeDtypeStruct((BH, D), jnp.float32),
)


def kernel(indices, table):
    tab_pad = jnp.zeros((KP, D), jnp.bfloat16).at[:VOCAB].set(
        table.astype(jnp.bfloat16))
    c0 = _hist(indices[:BH])
    c1 = _hist(indices[BH:])
    o0 = _mm(c0, tab_pad)
    o1 = _mm(c1, tab_pad)
    return jnp.concatenate([o0, o1], axis=0)


# speedup vs baseline: 1.2356x; 1.2356x over previous
import functools

import jax
import jax.numpy as jnp
from jax import lax
from jax.experimental import pallas as pl
from jax.experimental.pallas import tpu as pltpu
from jax.experimental.pallas import tpu_sc as plsc

B = 16384
L = 200
VOCAB = 1000
D = 64
KP = 1024

NC = 2
NS = 16
NW = NC * NS
ROWS_PER_W = B // NW
CHUNK = 32
NCHUNKS = ROWS_PER_W // CHUNK
NPAIR = NCHUNKS // 2

_mesh = plsc.VectorSubcoreMesh(core_axis_name="c", subcore_axis_name="s")


@functools.partial(
    pl.kernel,
    mesh=_mesh,
    out_type=jax.ShapeDtypeStruct((B, KP), jnp.float32),
    scratch_types=[
        pltpu.VMEM((2, CHUNK, L), jnp.int32),
        pltpu.VMEM((2, CHUNK, KP), jnp.float32),
        pltpu.VMEM_SHARED((CHUNK, KP), jnp.float32),
        pltpu.SemaphoreType.DMA,
        pltpu.SemaphoreType.DMA,
        pltpu.SemaphoreType.DMA,
        pltpu.SemaphoreType.DMA,
        pltpu.SemaphoreType.DMA,
        pltpu.SemaphoreType.DMA,
    ],
    compiler_params=pltpu.CompilerParams(needs_layout_passes=False),
)
def _hist(idx_hbm, cnt_hbm, idx_v, cnt_v, zshared,
          osem0, osem1, zsem0, zsem1, isem0, isem1):
    wid = lax.axis_index("s") * NC + lax.axis_index("c")
    row_base_w = wid * ROWS_PER_W
    zero16 = jnp.zeros((16,), jnp.float32)
    ones = jnp.full((16,), 1.0, jnp.float32)
    elig = lax.iota(jnp.int32, 16) >= 8
    osems = (osem0, osem1)
    zsems = (zsem0, zsem1)
    isems = (isem0, isem1)

    _OFFS = tuple(s * 16 for s in range(L // 16)) + (L - 16,)

    def scatter_chunk(b):
        bvec = jnp.full((16,), b, jnp.int32)

        def row_body(r, carry2):
            rvec = jnp.full((16,), 0, jnp.int32) + r
            for off in _OFFS[:-1]:
                ivec = idx_v[b, r, pl.ds(off, 16)]
                plsc.addupdate_scatter(cnt_v, [bvec, rvec, ivec], ones)
            ivec = idx_v[b, r, pl.ds(_OFFS[-1], 16)]
            plsc.addupdate_scatter(cnt_v, [bvec, rvec, ivec], ones,
                                   mask=elig)
            return carry2

        lax.fori_loop(0, CHUNK, row_body, 0)

    def start_idx(ci, b):
        cic = jnp.minimum(ci, NCHUNKS - 1)
        base = row_base_w + cic * CHUNK
        pltpu.async_copy(idx_hbm.at[pl.ds(base, CHUNK)], idx_v.at[b],
                         isems[b])

    def wait_idx(ci, b):
        cic = jnp.minimum(ci, NCHUNKS - 1)
        base = row_base_w + cic * CHUNK
        pltpu.make_async_copy(idx_hbm.at[pl.ds(base, CHUNK)], idx_v.at[b],
                              isems[b]).wait()

    def start_zero(b):
        pltpu.async_copy(zshared, cnt_v.at[b], zsems[b])

    def wait_zero(b):
        pltpu.make_async_copy(zshared, cnt_v.at[b], zsems[b]).wait()

    def start_out(ci, b):
        base = row_base_w + ci * CHUNK
        pltpu.async_copy(cnt_v.at[b], cnt_hbm.at[pl.ds(base, CHUNK)],
                         osems[b])

    def wait_out(ci, b):
        base = row_base_w + ci * CHUNK
        pltpu.make_async_copy(cnt_v.at[b],
                              cnt_hbm.at[pl.ds(base, CHUNK)],
                              osems[b]).wait()

    def clear_body(i, c):
        for v in range(16):
            cnt_v[0, i // (KP // 256),
                  pl.ds((i % (KP // 256) * 16 + v) * 16, 16)] = zero16
        return c

    lax.fori_loop(0, CHUNK * KP // 256, clear_body, 0)

    @pl.when(lax.axis_index("s") == 0)
    def _publish():
        pltpu.sync_copy(cnt_v.at[0], zshared)

    plsc.subcore_barrier()
    start_zero(1)
    start_idx(0, 0)
    start_idx(1, 1)

    wait_idx(0, 0)
    scatter_chunk(0)
    start_out(0, 0)
    start_idx(2, 0)
    wait_zero(1)
    wait_idx(1, 1)
    scatter_chunk(1)
    start_out(1, 1)
    start_idx(3, 1)

    def pair_body(g, carry):
        for b in range(2):
            ci = g * 2 + b
            wait_out(ci - 2, b)
            start_zero(b)
        for b in range(2):
            ci = g * 2 + b
            wait_zero(b)
            wait_idx(ci, b)
            scatter_chunk(b)
            start_out(ci, b)
            start_idx(ci + 2, b)
        return carry

    lax.fori_loop(1, NPAIR, pair_body, 0)
    for b in range(2):
        wait_out(NCHUNKS - 2 + b, b)


TM = 2048


def _mm_body(c_ref, t_ref, o_ref):
    o_ref[...] = jnp.dot(
        c_ref[...].astype(jnp.bfloat16),
        t_ref[...],
        preferred_element_type=jnp.float32,
    )


_mm = pl.pallas_call(
    _mm_body,
    grid=(B // TM,),
    in_specs=[
        pl.BlockSpec((TM, KP), lambda i: (i, 0)),
        pl.BlockSpec((KP, D), lambda i: (0, 0)),
    ],
    out_specs=pl.BlockSpec((TM, D), lambda i: (i, 0)),
    out_shape=jax.ShapeDtypeStruct((B, D), jnp.float32),
)


def kernel(indices, table):
    counts = _hist(indices)
    tab_pad = jnp.zeros((KP, D), jnp.bfloat16).at[:VOCAB].set(
        table.astype(jnp.bfloat16))
    return _mm(counts, tab_pad)


# speedup vs baseline: 1.4673x; 1.1875x over previous
import functools

import jax
import jax.numpy as jnp
from jax import lax
from jax.experimental import pallas as pl
from jax.experimental.pallas import tpu as pltpu
from jax.experimental.pallas import tpu_sc as plsc

B = 16384
L = 200
VOCAB = 1000
D = 64
KP = 1024
KPW = KP // 4

NC = 2
NS = 16
NW = NC * NS
ROWS_PER_W = B // NW
CHUNK = 32
NCHUNKS = ROWS_PER_W // CHUNK
NPAIR = NCHUNKS // 2

_mesh = plsc.VectorSubcoreMesh(core_axis_name="c", subcore_axis_name="s")


@functools.partial(
    pl.kernel,
    mesh=_mesh,
    out_type=jax.ShapeDtypeStruct((B, KPW), jnp.int32),
    scratch_types=[
        pltpu.VMEM((2, CHUNK, L), jnp.int32),
        pltpu.VMEM((2, CHUNK, KPW), jnp.int32),
        pltpu.VMEM_SHARED((CHUNK, KPW), jnp.int32),
        pltpu.SemaphoreType.DMA,
        pltpu.SemaphoreType.DMA,
        pltpu.SemaphoreType.DMA,
        pltpu.SemaphoreType.DMA,
        pltpu.SemaphoreType.DMA,
        pltpu.SemaphoreType.DMA,
    ],
    compiler_params=pltpu.CompilerParams(needs_layout_passes=False),
)
def _hist(idx_hbm, cnt_hbm, idx_v, cnt_v, zshared,
          osem0, osem1, zsem0, zsem1, isem0, isem1):
    wid = lax.axis_index("s") * NC + lax.axis_index("c")
    row_base_w = wid * ROWS_PER_W
    zero16 = jnp.zeros((16,), jnp.int32)
    elig = lax.iota(jnp.int32, 16) >= 8
    osems = (osem0, osem1)
    zsems = (zsem0, zsem1)
    isems = (isem0, isem1)

    _OFFS = tuple(s * 16 for s in range(L // 16)) + (L - 16,)

    def scatter_chunk(b):
        bvec = jnp.full((16,), b, jnp.int32)

        one = jnp.full((16,), 1, jnp.int32)

        def row_body(r, carry2):
            rvec = jnp.full((16,), 0, jnp.int32) + r
            for off in _OFFS[:-1]:
                ivec = idx_v[b, r, pl.ds(off, 16)]
                wvec = lax.shift_right_logical(ivec, 2)
                bval = one << ((ivec & 3) << 3)
                plsc.addupdate_scatter(cnt_v, [bvec, rvec, wvec], bval)
            ivec = idx_v[b, r, pl.ds(_OFFS[-1], 16)]
            wvec = lax.shift_right_logical(ivec, 2)
            bval = one << ((ivec & 3) << 3)
            plsc.addupdate_scatter(cnt_v, [bvec, rvec, wvec], bval,
                                   mask=elig)
            return carry2

        lax.fori_loop(0, CHUNK, row_body, 0)

    def start_idx(ci, b):
        cic = jnp.minimum(ci, NCHUNKS - 1)
        base = row_base_w + cic * CHUNK
        pltpu.async_copy(idx_hbm.at[pl.ds(base, CHUNK)], idx_v.at[b],
                         isems[b])

    def wait_idx(ci, b):
        cic = jnp.minimum(ci, NCHUNKS - 1)
        base = row_base_w + cic * CHUNK
        pltpu.make_async_copy(idx_hbm.at[pl.ds(base, CHUNK)], idx_v.at[b],
                              isems[b]).wait()

    def start_zero(b):
        pltpu.async_copy(zshared, cnt_v.at[b], zsems[b])

    def wait_zero(b):
        pltpu.make_async_copy(zshared, cnt_v.at[b], zsems[b]).wait()

    def start_out(ci, b):
        base = row_base_w + ci * CHUNK
        pltpu.async_copy(cnt_v.at[b], cnt_hbm.at[pl.ds(base, CHUNK)],
                         osems[b])

    def wait_out(ci, b):
        base = row_base_w + ci * CHUNK
        pltpu.make_async_copy(cnt_v.at[b],
                              cnt_hbm.at[pl.ds(base, CHUNK)],
                              osems[b]).wait()

    def clear_body(i, c):
        for v in range(KPW // 16):
            cnt_v[0, i, pl.ds(v * 16, 16)] = zero16
        return c

    lax.fori_loop(0, CHUNK, clear_body, 0)

    @pl.when(lax.axis_index("s") == 0)
    def _publish():
        pltpu.sync_copy(cnt_v.at[0], zshared)

    plsc.subcore_barrier()
    start_zero(1)
    start_idx(0, 0)
    start_idx(1, 1)

    wait_idx(0, 0)
    scatter_chunk(0)
    start_out(0, 0)
    start_idx(2, 0)
    wait_zero(1)
    wait_idx(1, 1)
    scatter_chunk(1)
    start_out(1, 1)
    start_idx(3, 1)

    def pair_body(g, carry):
        for b in range(2):
            ci = g * 2 + b
            wait_out(ci - 2, b)
            start_zero(b)
        for b in range(2):
            ci = g * 2 + b
            wait_zero(b)
            wait_idx(ci, b)
            scatter_chunk(b)
            start_out(ci, b)
            start_idx(ci + 2, b)
        return carry

    lax.fori_loop(1, NPAIR, pair_body, 0)
    for b in range(2):
        wait_out(NCHUNKS - 2 + b, b)


TM = 2048


def _mm_body(c_ref, t_ref, o_ref):
    c = c_ref[...]
    acc = jnp.zeros(o_ref.shape, jnp.float32)
    for k in range(4):
        bk = ((c >> (8 * k)) & 0xFF).astype(jnp.bfloat16)
        acc += jnp.dot(bk, t_ref[k * KPW:(k + 1) * KPW, :],
                       preferred_element_type=jnp.float32)
    o_ref[...] = acc


_mm = pl.pallas_call(
    _mm_body,
    grid=(B // TM,),
    in_specs=[
        pl.BlockSpec((TM, KPW), lambda i: (i, 0)),
        pl.BlockSpec((KP, D), lambda i: (0, 0)),
    ],
    out_specs=pl.BlockSpec((TM, D), lambda i: (i, 0)),
    out_shape=jax.ShapeDtypeStruct((B, D), jnp.float32),
)


def kernel(indices, table):
    counts = _hist(indices)
    tab_pad = jnp.zeros((KP, D), jnp.float32).at[:VOCAB].set(table)
    tab_perm = jnp.concatenate(
        [tab_pad[k::4] for k in range(4)], axis=0).astype(jnp.bfloat16)
    return _mm(counts, tab_perm)
